# baseline (speedup 1.0000x reference)
import functools

import jax
import jax.numpy as jnp
from jax import lax
from jax.experimental import pallas as pl
from jax.experimental.pallas import tpu as pltpu
from jax.experimental.pallas import tpu_sc as plsc

_N = 16384
_D = 2048
_TEMP = 0.1
_THRESH = 0.85
_K = 100
_EPS = 1e-8

_NS = 8192
_NT = _N - _NS
_BLK = 1024
_TSTEPS = _NT // _BLK

_NC = 2
_NSUB = 16
_NW = _NC * _NSUB
_RPW = _NS // _NW
_CH = 8
_NBUF = 4
_NCH = _RPW // _CH
_LANE = 16


def _tc_body(qcol_ref, pblk_ref, o_ref):
    qcol = qcol_ref[...]
    blk = pblk_ref[...]
    dotq = lax.dot_general(blk, qcol, (((1,), (0,)), ((), ())),
                           preferred_element_type=jnp.float32)
    sq = blk * blk
    ones = jnp.ones((_D, 1), dtype=jnp.float32)
    ssq = lax.dot_general(sq, ones, (((1,), (0,)), ((), ())),
                          preferred_element_type=jnp.float32)
    qn = jnp.maximum(jnp.sqrt(jnp.sum(qcol * qcol)), _EPS)
    pn = jnp.maximum(jnp.sqrt(ssq), _EPS)
    s = (dotq / (pn * qn)) / _TEMP
    o_ref[...] = s.reshape(_BLK // 128, 128)


def _tc_sims(qcol, patterns):
    return pl.pallas_call(
        _tc_body,
        grid=(_TSTEPS,),
        in_specs=[
            pl.BlockSpec((_D, 1), lambda i: (0, 0)),
            pl.BlockSpec((_BLK, _D), lambda i: (i, 0)),
        ],
        out_specs=pl.BlockSpec((_BLK // 128, 128), lambda i: (i, 0)),
        out_shape=jax.ShapeDtypeStruct((_NT // 128, 128), jnp.float32),
    )(qcol, patterns)


_DN = lax.GatherDimensionNumbers(offset_dims=(), collapsed_slice_dims=(0,),
                                 start_index_map=(0,))


def _shuf(v, idx):
    return lax.gather(v, idx[:, None], _DN, (1,),
                      mode=lax.GatherScatterMode.PROMISE_IN_BOUNDS)


def _lanesum(v):
    lanes = lax.iota(jnp.int32, _LANE)
    for sh in (8, 4, 2, 1):
        v = v + _shuf(v, lanes ^ sh)
    return v


def _sc_body(q_hbm, pat_hbm, dot_hbm, ssq_hbm,
             qv, b0, b1, b2, b3, dots_v, ssqs_v,
             qsem, s0, s1, s2, s3):
    bufs = (b0, b1, b2, b3)
    sems = (s0, s1, s2, s3)
    wid = lax.axis_index("s") * _NC + lax.axis_index("c")
    base = _NT + wid * _RPW

    pltpu.async_copy(q_hbm, qv, qsem).wait()
    for b in range(_NBUF):
        pltpu.async_copy(pat_hbm.at[pl.ds(base + b * _CH, _CH)], bufs[b], sems[b])

    lanes = lax.iota(jnp.int32, _LANE)
    zz = jnp.zeros((_LANE,), jnp.float32)

    def chunk_results(buf):
        accs = [jnp.zeros((_LANE,), jnp.float32) for _ in range(2 * _CH)]

        def col_body(c, accs):
            accs = list(accs)
            qc = qv[pl.ds(c * _LANE, _LANE)]
            for r in range(_CH):
                v = buf[r, pl.ds(c * _LANE, _LANE)]
                accs[2 * r] = accs[2 * r] + v * qc
                accs[2 * r + 1] = accs[2 * r + 1] + v * v
            return tuple(accs)

        accs = lax.fori_loop(0, _D // _LANE, col_body, tuple(accs))
        drow = [_lanesum(accs[2 * r]) for r in range(_CH)]
        srow = [_lanesum(accs[2 * r + 1]) for r in range(_CH)]
        return drow, srow

    def group(g, _):
        dsc, ssc = [], []
        for b in range(_NBUF):
            j = g * _NBUF + b
            pltpu.make_async_copy(
                pat_hbm.at[pl.ds(0, _CH)], bufs[b], sems[b]).wait()
            drow, srow = chunk_results(bufs[b])
            dsc += drow
            ssc += srow
            jn = j + _NBUF

            @pl.when(jn < _NCH)
            def _():
                pltpu.async_copy(
                    pat_hbm.at[pl.ds(base + jn * _CH, _CH)], bufs[b], sems[b])

        for h in range(2):
            dvec, svec = zz, zz
            for r in range(_LANE):
                sel = lanes == r
                dvec = jnp.where(sel, dsc[h * _LANE + r], dvec)
                svec = jnp.where(sel, ssc[h * _LANE + r], svec)
            off = g * _NBUF * _CH + h * _LANE
            dots_v[pl.ds(off, _LANE)] = dvec
            ssqs_v[pl.ds(off, _LANE)] = svec
        return 0

    lax.fori_loop(0, _NCH // _NBUF, group, 0)

    pltpu.sync_copy(dots_v, dot_hbm.at[pl.ds(wid * _RPW, _RPW)])
    pltpu.sync_copy(ssqs_v, ssq_hbm.at[pl.ds(wid * _RPW, _RPW)])


def _sc_dots(query_pattern, patterns):
    mesh = plsc.VectorSubcoreMesh(core_axis_name="c", subcore_axis_name="s")
    f = functools.partial(
        pl.kernel,
        mesh=mesh,
        out_type=[
            jax.ShapeDtypeStruct((_NS,), jnp.float32),
            jax.ShapeDtypeStruct((_NS,), jnp.float32),
        ],
        scratch_types=[
            pltpu.VMEM((_D,), jnp.float32),
        ] + [pltpu.VMEM((_CH, _D), jnp.float32) for _ in range(_NBUF)] + [
            pltpu.VMEM((_RPW,), jnp.float32),
            pltpu.VMEM((_RPW,), jnp.float32),
        ] + [pltpu.SemaphoreType.DMA for _ in range(_NBUF + 1)],
    )(_sc_body)
    return f(query_pattern, patterns)


def _fin_body(qcol_ref, slo_ref, dhi_ref, shi_ref, pany_ref,
              outp_ref, outv_ref, sims_sc, sem):
    qcol = qcol_ref[...]
    qn = jnp.maximum(jnp.sqrt(jnp.sum(qcol * qcol)), _EPS)
    sims_sc[pl.ds(0, _NT // 128), :] = slo_ref[...]
    pn = jnp.maximum(jnp.sqrt(shi_ref[...]), _EPS)
    sims_sc[pl.ds(_NT // 128, _NS // 128), :] = (dhi_ref[...] / (pn * qn)) / _TEMP

    outp_ref[...] = jnp.zeros_like(outp_ref)
    outv_ref[...] = jnp.zeros_like(outv_ref)
    lin = (lax.broadcasted_iota(jnp.int32, (128, 128), 0) * 128
           + lax.broadcasted_iota(jnp.int32, (128, 128), 1))
    lid = lax.broadcasted_iota(jnp.int32, (1, 128), 1)
    big = jnp.int32(2 ** 30)

    m0 = jnp.max(sims_sc[...])

    def cond(c):
        k, m = c
        return (k < _K) & (m >= _THRESH)

    def body(c):
        k, m = c
        s = sims_sc[...]
        idx = jnp.min(jnp.where(s == m, lin, big))
        outv_ref[...] = jnp.where(lid == k, m, outv_ref[...])
        sims_sc[...] = jnp.where(lin == idx, -jnp.inf, s)
        cp = pltpu.make_async_copy(
            pany_ref.at[pl.ds(idx, 1)], outp_ref.at[pl.ds(k, 1)], sem)
        cp.start()
        cp.wait()
        return k + 1, jnp.max(sims_sc[...])

    lax.while_loop(cond, body, (jnp.int32(0), m0))


def _finish(qcol, sims_lo, dot_hi, ssq_hi, patterns):
    return pl.pallas_call(
        _fin_body,
        grid=(1,),
        in_specs=[
            pl.BlockSpec((_D, 1), lambda i: (0, 0)),
            pl.BlockSpec((_NT // 128, 128), lambda i: (0, 0)),
            pl.BlockSpec((_NS // 128, 128), lambda i: (0, 0)),
            pl.BlockSpec((_NS // 128, 128), lambda i: (0, 0)),
            pl.BlockSpec(memory_space=pl.ANY),
        ],
        out_specs=[
            pl.BlockSpec((128, _D), lambda i: (0, 0)),
            pl.BlockSpec((1, 128), lambda i: (0, 0)),
        ],
        out_shape=[
            jax.ShapeDtypeStruct((128, _D), jnp.float32),
            jax.ShapeDtypeStruct((1, 128), jnp.float32),
        ],
        scratch_shapes=[
            pltpu.VMEM((128, 128), jnp.float32),
            pltpu.SemaphoreType.DMA,
        ],
    )(qcol, sims_lo, dot_hi.reshape(_NS // 128, 128),
      ssq_hi.reshape(_NS // 128, 128), patterns)


@jax.jit
def _run(query_pattern, patterns):
    qcol = query_pattern.reshape(_D, 1)
    dot_hi, ssq_hi = _sc_dots(query_pattern, patterns)
    sims_lo = _tc_sims(qcol, patterns)
    outp, outv = _finish(qcol, sims_lo, dot_hi, ssq_hi, patterns)
    return outp[:_K], outv[0, :_K]


def kernel(query_pattern, patterns):
    return _run(query_pattern, patterns)
